# baseline (speedup 1.0000x reference)
import jax
import jax.numpy as jnp
from jax.experimental import pallas as pl

NUM_CLASSES = 22
H_IMG, W_IMG = 256, 320
VOTE_STEPS = 160
VOTE_THRESH = 0.5


def _conv_t(x, w, b):
    k = jnp.flip(w, (2, 3)).transpose(1, 0, 2, 3)
    y = jax.lax.conv_general_dilated(x, k, (1, 1), ((2, 2), (2, 2)),
                                     lhs_dilation=(2, 2),
                                     dimension_numbers=('NCHW', 'OIHW', 'NCHW'))
    return y + b[None, :, None, None]


def _hough_vote(seg2d, dirs, depth2d, fx, fy, px, py):
    H, W = seg2d.shape
    n = dirs / (jnp.linalg.norm(dirs, axis=-1, keepdims=True) + 1e-6)
    u = n[..., 0].reshape(-1)
    v = n[..., 1].reshape(-1)
    ys, xs = jnp.meshgrid(jnp.arange(H, dtype=jnp.float32),
                          jnp.arange(W, dtype=jnp.float32), indexing='ij')
    xs = xs.reshape(-1); ys = ys.reshape(-1)
    w = (seg2d * (seg2d > VOTE_THRESH)).reshape(-1)

    def step(acc, t):
        xt = jnp.round(xs + t * u).astype(jnp.int32)
        yt = jnp.round(ys + t * v).astype(jnp.int32)
        inb = (xt >= 0) & (xt < W) & (yt >= 0) & (yt < H)
        idx = jnp.clip(yt, 0, H - 1) * W + jnp.clip(xt, 0, W - 1)
        return acc.at[idx].add(jnp.where(inb, w, 0.0)), None

    acc, _ = jax.lax.scan(step, jnp.zeros(H * W, jnp.float32),
                          jnp.arange(1, VOTE_STEPS + 1, dtype=jnp.float32))
    vote_map = acc.reshape(H, W)
    ci = jnp.argmax(acc)
    cy = (ci // W).astype(jnp.float32)
    cx = (ci % W).astype(jnp.float32)
    mask = (seg2d > VOTE_THRESH).astype(jnp.float32)
    tz = jnp.sum(depth2d * mask) / (jnp.sum(mask) + 1e-6)
    tx = (cx - px) * tz / fx
    ty = (cy - py) * tz / fy
    return vote_map, jnp.stack([cx, cy]), jnp.stack([tx, ty, tz])


def kernel(x1, x2, x3, x4, x5, depth, seg, fx, fy, px, py,
           w5, b5, w4, b4, w3, b3, w2, b2, w1, b1, wv, bv, wc, bc):
    x = jax.nn.relu(_conv_t(x1, w5, b5))
    x = jax.nn.relu(_conv_t(jnp.concatenate([x, x2], 1), w4, b4))
    x = jax.nn.relu(_conv_t(jnp.concatenate([x, x3], 1), w3, b3))
    x = jax.nn.relu(_conv_t(jnp.concatenate([x, x4], 1), w2, b2))
    x = jax.nn.relu(_conv_t(jnp.concatenate([x, x5], 1), w1, b1))
    bottom_vertex = jnp.einsum('bchw,oc->bohw', x, wv) + bv[None, :, None, None]
    logits = jnp.einsum('bchw,oc->bohw', x, wc) + bc[None, :, None, None]
    bottom_label = jnp.argmax(logits, axis=1)
    center_dirs = jnp.transpose(bottom_vertex[:, :2], (0, 2, 3, 1))
    seg_fused = jnp.clip(jnp.sum(seg, axis=1), 0.0, 1.0)
    vote_maps, centers, translations = jax.vmap(
        _hough_vote, in_axes=(0, 0, 0, None, None, None, None)
    )(seg_fused, center_dirs, depth[:, 0], fx, fy, px, py)
    return bottom_vertex, bottom_label, vote_maps, centers, translations


# speedup vs baseline: 1.0598x; 1.0598x over previous
import jax
import jax.numpy as jnp
from jax.experimental import pallas as pl
from jax.experimental.pallas import tpu as pltpu

_CP = getattr(pltpu, "CompilerParams", None) or getattr(pltpu, "TPUCompilerParams")
_H, _W = 256, 320
_STEPS = 160
_INTERP = False


def _convt_pallas(x, w, b):
    B, H, W, Cin = x.shape
    Cout = w.shape[1]
    Wp = ((W + 7) // 8) * 8
    xp = jnp.pad(x, ((0, 0), (1, 1), (1, Wp - W + 1), (0, 0)))
    xps = jnp.stack([jnp.stack([xp[:, a:a + H + 1, bb:bb + Wp + 1, :]
                                for bb in range(2)], 1) for a in range(2)], 1)
    wm = jnp.stack([w[:, :, 3 - a - 2 * dy, 3 - bb - 2 * dx]
                    for a in range(2) for bb in range(2)
                    for dy in range(2) for dx in range(2)])
    wm = wm.reshape(2, 2, 4, Cin, Cout)

    def kern(xp_ref, wm_ref, b_ref, o_ref):
        acc = jnp.zeros((H * Wp, Cout), jnp.float32)
        for k, (dy, dx) in enumerate(((0, 0), (0, 1), (1, 0), (1, 1))):
            sl = xp_ref[0, 0, 0, dy:dy + H, dx:dx + Wp, :].reshape(H * Wp, Cin)
            acc = acc + jnp.dot(sl, wm_ref[0, 0, k],
                                preferred_element_type=jnp.float32)
        acc = jnp.maximum(acc + b_ref[0], 0.0)
        o_ref[0, 0, 0] = acc.reshape(H, Wp, Cout)

    out = pl.pallas_call(
        kern,
        grid=(B, 2, 2),
        in_specs=[
            pl.BlockSpec((1, 1, 1, H + 1, Wp + 1, Cin),
                         lambda bi, a, bb: (bi, a, bb, 0, 0, 0)),
            pl.BlockSpec((1, 1, 4, Cin, Cout), lambda bi, a, bb: (a, bb, 0, 0, 0)),
            pl.BlockSpec((1, Cout), lambda bi, a, bb: (0, 0)),
        ],
        out_specs=pl.BlockSpec((1, 1, 1, H, Wp, Cout),
                               lambda bi, a, bb: (bi, a, bb, 0, 0, 0)),
        out_shape=jax.ShapeDtypeStruct((B, 2, 2, H, Wp, Cout), jnp.float32),
        compiler_params=_CP(dimension_semantics=("parallel", "arbitrary", "arbitrary"),
                            vmem_limit_bytes=100 * 1024 * 1024),
        interpret=_INTERP,
    )(xps, wm, b.reshape(1, Cout))
    y = out.transpose(0, 3, 1, 4, 2, 5).reshape(B, 2 * H, 2 * Wp, Cout)
    return y[:, :, :2 * W, :]


def _heads_pallas(h, seg, wv, bv, wc, bc):
    B = h.shape[0]
    wcat = jnp.zeros((64, 128), jnp.float32)
    wcat = wcat.at[:, :66].set(wv.T).at[:, 66:88].set(wc.T)
    bcat = jnp.zeros((1, 128), jnp.float32)
    bcat = bcat.at[0, :66].set(bv).at[0, 66:88].set(bc)

    def kern(h_ref, seg_ref, w_ref, b_ref, feat_ref, lab_ref, segf_ref):
        x2 = h_ref[0].reshape(64 * 320, 64)
        y = jnp.dot(x2, w_ref[...], preferred_element_type=jnp.float32) + b_ref[...]
        y3 = y.reshape(64, 320, 128)
        feat_ref[0] = y3
        lanes = jax.lax.broadcasted_iota(jnp.int32, (64, 320, 128), 2)
        ym = jnp.where((lanes >= 66) & (lanes < 88), y3, -jnp.inf)
        lab_ref[0] = jnp.argmax(ym, axis=-1).astype(jnp.int32) - 66
        segf_ref[0] = jnp.clip(jnp.sum(seg_ref[0], axis=0), 0.0, 1.0)

    feat, lab, segf = pl.pallas_call(
        kern,
        grid=(B, 4),
        in_specs=[
            pl.BlockSpec((1, 64, 320, 64), lambda bi, r: (bi, r, 0, 0)),
            pl.BlockSpec((1, 22, 64, 320), lambda bi, r: (bi, 0, r, 0)),
            pl.BlockSpec((64, 128), lambda bi, r: (0, 0)),
            pl.BlockSpec((1, 128), lambda bi, r: (0, 0)),
        ],
        out_specs=[
            pl.BlockSpec((1, 64, 320, 128), lambda bi, r: (bi, r, 0, 0)),
            pl.BlockSpec((1, 64, 320), lambda bi, r: (bi, r, 0)),
            pl.BlockSpec((1, 64, 320), lambda bi, r: (bi, r, 0)),
        ],
        out_shape=[
            jax.ShapeDtypeStruct((B, 256, 320, 128), jnp.float32),
            jax.ShapeDtypeStruct((B, 256, 320), jnp.int32),
            jax.ShapeDtypeStruct((B, 256, 320), jnp.float32),
        ],
        compiler_params=_CP(dimension_semantics=("parallel", "arbitrary"),
                            vmem_limit_bytes=100 * 1024 * 1024),
        interpret=_INTERP,
    )(h, seg, wcat, bcat)
    return feat, lab, segf


_DUMP = 81920
_ROWS = 81928


def _hough_pallas(segf4, dirx4, diry4, dep4, cam):
    B = segf4.shape[0]

    def kern(cam_ref, sf_ref, dx_ref, dy_ref, dep_ref,
             votes_ref, cent_ref, trans_ref,
             a0, a1, a2, a3, a4, a5, a6, a7, lut, idxv, metav, wv_, idxs, metas, ws_, sems):
        accs = (a0, a1, a2, a3, a4, a5, a6, a7)

        def zblk(i, _):
            z8 = jnp.zeros((8, 128), jnp.float32)
            for a in accs:
                a[pl.ds(8 * i, 8), 0, :] = z8
            return 0
        jax.lax.fori_loop(0, 81, zblk, 0)
        sub8 = jax.lax.broadcasted_iota(jnp.int32, (8, 128), 0)
        lan8 = jax.lax.broadcasted_iota(jnp.int32, (8, 128), 1)
        for g8 in range(16):
            lut[pl.ds(8 * g8, 8), 0, :] = jnp.where(sub8 + 8 * g8 == lan8, 1.0, 0.0)

        xs = jax.lax.broadcasted_iota(jnp.int32, (1, 320), 1).astype(jnp.float32)

        def compute_chunk(row, slot):
            un = dx_ref[0, row]
            vn = dy_ref[0, row]
            ss = sf_ref[0, row]
            nr0 = jnp.sqrt(un * un + vn * vn)
            nrm = nr0 + 1e-6
            u = un / nrm
            v = vn / nrm
            w = jnp.where(ss > 0.5, ss, 0.0)
            voter = ss > 0.5
            safe = nr0 >= 1e-5
            ys = jnp.full((1, 320), row, jnp.int32).astype(jnp.float32)
            tmin = jnp.full((8, 320), 999, jnp.int32)
            tmax = jnp.full((8, 320), -1, jnp.int32)
            for tb in range(20):
                trow = jax.lax.broadcasted_iota(jnp.int32, (8, 320), 0) + 8 * tb
                t = trow.astype(jnp.float32) + 1.0
                xt = jnp.round(xs + t * u).astype(jnp.int32)
                yt = jnp.round(ys + t * v).astype(jnp.int32)
                inb = ((xt >= 0) & (xt < 320)) & ((yt >= 0) & (yt < 256))
                idxq = jnp.where(inb, yt * 320 + xt, _DUMP)
                idxv[slot, 8 * tb:8 * tb + 8, :] = idxq
                tmin = jnp.minimum(tmin, jnp.where(inb, trow, 999))
                tmax = jnp.maximum(tmax, jnp.where(inb, trow, -1))
            idxv[slot, 160:168, :] = jnp.full((8, 320), _DUMP, jnp.int32)
            idxv[slot, 168:176, :] = jnp.full((8, 320), _DUMP, jnp.int32)
            tlo = jnp.min(tmin, axis=0, keepdims=True)
            thi = jnp.max(tmax, axis=0, keepdims=True) + 1
            tlo = jnp.where(voter, tlo, 0)
            thi = jnp.where(voter, thi, 0)
            metav[slot, 0:1, :] = tlo
            metav[slot, 1:2, :] = thi
            metav[slot, 2:3, :] = jnp.where(safe, 1, 0)
            wv_[slot] = w
            pltpu.make_async_copy(idxv.at[slot], idxs.at[slot], sems.at[slot, 0]).start()
            pltpu.make_async_copy(metav.at[slot], metas.at[slot], sems.at[slot, 1]).start()
            pltpu.make_async_copy(wv_.at[slot], ws_.at[slot], sems.at[slot, 2]).start()

        def wait_chunk(slot):
            pltpu.make_async_copy(idxv.at[slot], idxs.at[slot], sems.at[slot, 0]).wait()
            pltpu.make_async_copy(metav.at[slot], metas.at[slot], sems.at[slot, 1]).wait()
            pltpu.make_async_copy(wv_.at[slot], ws_.at[slot], sems.at[slot, 2]).wait()

        lanes128 = jax.lax.broadcasted_iota(jnp.int32, (1, 128), 1)

        def deposit_chunk(slot):
            def per_pixel(pix, _):
                tlo = metas[slot, 0, pix]
                thi = metas[slot, 1, pix]
                wvec = jnp.full((1, 128), ws_[slot, 0, pix], jnp.float32)
                ng = jnp.maximum(0, (thi - tlo + 7) >> 3)

                def grp(g, _):
                    base = tlo + 8 * g
                    for j in range(8):
                        ix = idxs[slot, base + j, pix]
                        c = lut[pl.ds(ix & 127, 1), 0, :] * wvec
                        r = ix >> 7
                        accs[j][pl.ds(r, 1), 0, :] = accs[j][pl.ds(r, 1), 0, :] + c
                    return 0
                jax.lax.fori_loop(0, ng, grp, 0)
                return 0
            jax.lax.fori_loop(0, 320, per_pixel, 0)

        compute_chunk(0, 0)

        def chunk_body(c, _):
            s = jax.lax.rem(c, 2)
            wait_chunk(s)

            @pl.when(c + 1 < 256)
            def _():
                compute_chunk(c + 1, jax.lax.rem(c + 1, 2))
            deposit_chunk(s)
            return 0
        jax.lax.fori_loop(0, 256, chunk_body, 0)

        def merge_blk(g, _):
            s01 = a0[pl.ds(8 * g, 8), 0, :] + a1[pl.ds(8 * g, 8), 0, :]
            s23 = a2[pl.ds(8 * g, 8), 0, :] + a3[pl.ds(8 * g, 8), 0, :]
            s45 = a4[pl.ds(8 * g, 8), 0, :] + a5[pl.ds(8 * g, 8), 0, :]
            s67 = a6[pl.ds(8 * g, 8), 0, :] + a7[pl.ds(8 * g, 8), 0, :]
            votes_ref[0, pl.ds(pl.multiple_of(8 * g, 8), 8), :] = (
                (s01 + s23) + (s45 + s67))
            return 0
        jax.lax.fori_loop(0, 80, merge_blk, 0)

        vv = votes_ref[0]
        m1 = jnp.max(vv, axis=0, keepdims=True)
        m = jnp.max(m1, axis=1, keepdims=True)
        fi = (jax.lax.broadcasted_iota(jnp.int32, (640, 128), 0) * 128
              + jax.lax.broadcasted_iota(jnp.int32, (640, 128), 1))
        big = jnp.int32(1 << 30)
        ci1 = jnp.min(jnp.where(vv == m, fi, big), axis=0, keepdims=True)
        ci = jnp.min(ci1, axis=1, keepdims=True)
        cif = ci.astype(jnp.float32)
        q = jnp.floor(cif * (1.0 / 320.0))
        q = jnp.where((q + 1.0) * 320.0 <= cif, q + 1.0, q)
        q = jnp.where(q * 320.0 > cif, q - 1.0, q)
        cy = q
        cx = cif - 320.0 * q
        segf2 = sf_ref[0, :, 0, :]
        dep2 = dep_ref[0, :, 0, :]
        mask = jnp.where(segf2 > 0.5, 1.0, 0.0)
        msum = jnp.sum(jnp.sum(mask, axis=1, keepdims=True), axis=0, keepdims=True)
        dsum = jnp.sum(jnp.sum(dep2 * mask, axis=1, keepdims=True), axis=0, keepdims=True)
        tz = dsum / (msum + 1e-6)
        fxv = jnp.full((1, 1), cam_ref[0], jnp.float32)
        fyv = jnp.full((1, 1), cam_ref[1], jnp.float32)
        pxv = jnp.full((1, 1), cam_ref[2], jnp.float32)
        pyv = jnp.full((1, 1), cam_ref[3], jnp.float32)
        tx = (cx - pxv) * tz / fxv
        ty = (cy - pyv) * tz / fyv
        lane = jax.lax.broadcasted_iota(jnp.int32, (1, 128), 1)
        cxb = jnp.broadcast_to(cx, (1, 128))
        cyb = jnp.broadcast_to(cy, (1, 128))
        cent_ref[0] = jnp.where(lane == 0, cxb, jnp.where(lane == 1, cyb, 0.0))
        txb = jnp.broadcast_to(tx, (1, 128))
        tyb = jnp.broadcast_to(ty, (1, 128))
        tzb = jnp.broadcast_to(tz, (1, 128))
        trans_ref[0] = jnp.where(lane == 0, txb,
                                 jnp.where(lane == 1, tyb,
                                           jnp.where(lane == 2, tzb, 0.0)))

    votes, cent, trans = pl.pallas_call(
        kern,
        grid=(B,),
        in_specs=[
            pl.BlockSpec(memory_space=pltpu.SMEM),
            pl.BlockSpec((1, 256, 1, 320), lambda bi: (bi, 0, 0, 0)),
            pl.BlockSpec((1, 256, 1, 320), lambda bi: (bi, 0, 0, 0)),
            pl.BlockSpec((1, 256, 1, 320), lambda bi: (bi, 0, 0, 0)),
            pl.BlockSpec((1, 256, 1, 320), lambda bi: (bi, 0, 0, 0)),
        ],
        out_specs=[
            pl.BlockSpec((1, 640, 128), lambda bi: (bi, 0, 0)),
            pl.BlockSpec((1, 1, 128), lambda bi: (bi, 0, 0)),
            pl.BlockSpec((1, 1, 128), lambda bi: (bi, 0, 0)),
        ],
        out_shape=[
            jax.ShapeDtypeStruct((B, 640, 128), jnp.float32),
            jax.ShapeDtypeStruct((B, 1, 128), jnp.float32),
            jax.ShapeDtypeStruct((B, 1, 128), jnp.float32),
        ],
        scratch_shapes=[
        ] + [pltpu.VMEM((648, 1, 128), jnp.float32)] * 8 + [
            pltpu.VMEM((128, 1, 128), jnp.float32),
            pltpu.VMEM((2, 176, 320), jnp.int32),
            pltpu.VMEM((2, 3, 320), jnp.int32),
            pltpu.VMEM((2, 1, 320), jnp.float32),
            pltpu.SMEM((2, 176, 320), jnp.int32),
            pltpu.SMEM((2, 3, 320), jnp.int32),
            pltpu.SMEM((2, 1, 320), jnp.float32),
            pltpu.SemaphoreType.DMA((2, 3)),
        ],
        compiler_params=_CP(dimension_semantics=("parallel",),
                            vmem_limit_bytes=60 * 1024 * 1024),
        interpret=_INTERP,
    )(cam, segf4, dirx4, diry4, dep4)
    return votes, cent, trans


def _hough_jax(segf, dirx, diry, depth2d, fx, fy, px, py):
    def one(seg2d, u2, v2, d2):
        H, W = seg2d.shape
        nrm = jnp.sqrt(u2 * u2 + v2 * v2) + 1e-6
        u = (u2 / nrm).reshape(-1)
        v = (v2 / nrm).reshape(-1)
        ys, xs = jnp.meshgrid(jnp.arange(H, dtype=jnp.float32),
                              jnp.arange(W, dtype=jnp.float32), indexing='ij')
        xs = xs.reshape(-1); ys = ys.reshape(-1)
        w = (seg2d * (seg2d > 0.5)).reshape(-1)

        def step(acc, t):
            xt = jnp.round(xs + t * u).astype(jnp.int32)
            yt = jnp.round(ys + t * v).astype(jnp.int32)
            inb = (xt >= 0) & (xt < W) & (yt >= 0) & (yt < H)
            idx = jnp.clip(yt, 0, H - 1) * W + jnp.clip(xt, 0, W - 1)
            return acc.at[idx].add(jnp.where(inb, w, 0.0)), None

        acc, _ = jax.lax.scan(step, jnp.zeros(H * W, jnp.float32),
                              jnp.arange(1, _STEPS + 1, dtype=jnp.float32))
        vote_map = acc.reshape(H, W)
        ci = jnp.argmax(acc)
        cy = (ci // W).astype(jnp.float32)
        cx = (ci % W).astype(jnp.float32)
        mask = (seg2d > 0.5).astype(jnp.float32)
        tz = jnp.sum(d2 * mask) / (jnp.sum(mask) + 1e-6)
        tx = (cx - px) * tz / fx
        ty = (cy - py) * tz / fy
        return vote_map, jnp.stack([cx, cy]), jnp.stack([tx, ty, tz])

    return jax.vmap(one)(segf, dirx, diry, depth2d)


def kernel(x1, x2, x3, x4, x5, depth, seg, fx, fy, px, py,
           w5, b5, w4, b4, w3, b3, w2, b2, w1, b1, wv, bv, wc, bc):
    nhwc = lambda t: t.transpose(0, 2, 3, 1)
    h = _convt_pallas(nhwc(x1), w5, b5)
    h = _convt_pallas(jnp.concatenate([h, nhwc(x2)], -1), w4, b4)
    h = _convt_pallas(jnp.concatenate([h, nhwc(x3)], -1), w3, b3)
    h = _convt_pallas(jnp.concatenate([h, nhwc(x4)], -1), w2, b2)
    h = _convt_pallas(jnp.concatenate([h, nhwc(x5)], -1), w1, b1)
    feat, label, segf = _heads_pallas(h, seg, wv, bv, wc, bc)
    bottom_vertex = feat[..., :66].transpose(0, 3, 1, 2)
    B = segf.shape[0]
    r4 = lambda t: t.reshape(B, 256, 1, 320)
    cam = jnp.stack([fx, fy, px, py]).astype(jnp.float32)
    votes, cent, trans = _hough_pallas(
        r4(segf), r4(feat[..., 0]), r4(feat[..., 1]), r4(depth[:, 0]), cam)
    vote_maps = votes.reshape(B, 256, 320)
    centers = cent[:, 0, :2]
    translations = trans[:, 0, :3]
    return bottom_vertex, label, vote_maps, centers, translations
